# baseline (speedup 1.0000x reference)
import dataclasses
import functools

import jax
import jax.numpy as jnp
import numpy as np
from jax import lax
from jax.experimental import pallas as pl
from jax.experimental.pallas import tpu as pltpu
from jax.experimental.pallas import tpu_sc as plsc


def _positional_encoding(length: int, depth: int) -> np.ndarray:
    half = depth // 2
    positions = np.arange(length)[:, np.newaxis]
    depths = np.arange(half)[np.newaxis, :] / half
    angle_rates = 1.0 / (10000.0 ** depths)
    angle_rads = positions * angle_rates
    return np.concatenate(
        [np.sin(angle_rads), np.cos(angle_rads)], axis=-1
    ).astype(np.float32)


_NC, _NS, _L = 2, 16, 16
_NW = _NC * _NS
_W = 128


def kernel(x, table):
    B, S = x.shape
    V, D = table.shape
    scale = float(np.sqrt(float(D)))
    pe = jnp.asarray(_positional_encoding(S, D))

    assert B % _W == 0 and D % _L == 0
    bblk_per_s = B // _W
    n_sgrp = _NW // bblk_per_s
    assert S % n_sgrp == 0
    per_w = S // n_sgrp
    assert per_w >= 4 and per_w % 3 == 2

    xi = x.astype(jnp.int32)

    mesh = plsc.VectorSubcoreMesh(core_axis_name="c", subcore_axis_name="s")

    cp = pltpu.CompilerParams()
    if "needs_layout_passes" in pltpu.CompilerParams.__dataclass_fields__:
        cp = dataclasses.replace(cp, needs_layout_passes=False)

    @functools.partial(
        pl.kernel,
        mesh=mesh,
        compiler_params=cp,
        out_type=jax.ShapeDtypeStruct((B, S * D), jnp.float32),
        scratch_types=[
            pltpu.VMEM((S, D), jnp.float32),
            pltpu.VMEM((_W, S), jnp.int32),
            pltpu.VMEM((per_w, _W), jnp.int32),
            pltpu.VMEM((_W, D), jnp.float32),
            pltpu.VMEM((_W, D), jnp.float32),
            pltpu.VMEM((_W, D), jnp.float32),
            pltpu.SemaphoreType.DMA,
            pltpu.SemaphoreType.DMA,
            pltpu.SemaphoreType.DMA,
            pltpu.SemaphoreType.DMA,
            pltpu.SemaphoreType.DMA,
            pltpu.SemaphoreType.DMA,
        ],
    )
    def k(x_hbm, table_hbm, pe_hbm, out_hbm,
          pe_v, xb_v, idx_v, r0, r1, r2, g0, g1, g2, o0, o1, o2):
        rows = (r0, r1, r2)
        gsem = (g0, g1, g2)
        osem = (o0, o1, o2)
        wid = lax.axis_index("s") * _NC + lax.axis_index("c")
        bblk = wid % bblk_per_s
        s0 = (wid // bblk_per_s) * per_w
        b_lo = pl.multiple_of(bblk * _W, _W)

        pltpu.sync_copy(pe_hbm, pe_v)
        pltpu.sync_copy(x_hbm.at[pl.ds(b_lo, _W)], xb_v)

        iota = lax.iota(jnp.int32, _L)

        @pl.loop(0, per_w)
        def _(t):
            cidx = jnp.zeros((_L,), jnp.int32) + (s0 + t)
            for j in range(_W // _L):
                vals = plsc.load_gather(xb_v, [iota + (_L * j), cidx])
                idx_v[t, pl.ds(_L * j, _L)] = vals

        def gather(t, b):
            return pltpu.make_async_copy(
                table_hbm.at[idx_v.at[t]], rows[b], gsem[b])

        def out_slot(t):
            return out_hbm.at[
                pl.ds(b_lo, _W),
                pl.ds(pl.multiple_of((s0 + t) * D, D), D),
            ]

        def writeback(t, b):
            return pltpu.make_async_copy(rows[b], out_slot(t), osem[b])

        def compute(t, b):
            r = rows[b]
            pe_regs = [
                pe_v[s0 + t, pl.ds(cc * _L, _L)] for cc in range(D // _L)
            ]

            @pl.loop(0, _W)
            def _(i):
                for cc in range(D // _L):
                    sl = pl.ds(cc * _L, _L)
                    r[i, sl] = r[i, sl] * scale + pe_regs[cc]

        def body(t, b, issue_next, first=False):
            gather(t, b).wait()
            compute(t, b)
            pb = (b + 2) % 3

            def _wait_prev():
                writeback(t - 1, pb).wait()

            if first:
                pl.when(t >= 1)(_wait_prev)
            else:
                _wait_prev()

            if issue_next:
                gather(t + 2, pb).start()
            writeback(t, b).start()

        gather(0, 0).start()
        gather(1, 1).start()

        @pl.loop(0, per_w - 2, step=3)
        def _(t):
            body(t, 0, True, first=True)
            body(t + 1, 1, True)
            body(t + 2, 2, True)

        body(per_w - 2, (per_w - 2) % 3, False)
        body(per_w - 1, (per_w - 1) % 3, False)
        writeback(per_w - 1, (per_w - 1) % 3).wait()

    out = k(xi, table, pe)
    return out.reshape(B, S, D)


# speedup vs baseline: 1.0134x; 1.0134x over previous
import dataclasses
import functools

import jax
import jax.numpy as jnp
import numpy as np
from jax import lax
from jax.experimental import pallas as pl
from jax.experimental.pallas import tpu as pltpu
from jax.experimental.pallas import tpu_sc as plsc


def _positional_encoding(length: int, depth: int) -> np.ndarray:
    half = depth // 2
    positions = np.arange(length)[:, np.newaxis]
    depths = np.arange(half)[np.newaxis, :] / half
    angle_rates = 1.0 / (10000.0 ** depths)
    angle_rads = positions * angle_rates
    return np.concatenate(
        [np.sin(angle_rads), np.cos(angle_rads)], axis=-1
    ).astype(np.float32)


_NC, _NS, _L = 2, 16, 16
_NW = _NC * _NS
_W = 128


def kernel(x, table):
    B, S = x.shape
    V, D = table.shape
    scale = float(np.sqrt(float(D)))
    pe = jnp.asarray(_positional_encoding(S, D))

    assert B % _W == 0 and D % _L == 0
    bblk_per_s = B // _W
    n_sgrp = _NW // bblk_per_s
    assert S % n_sgrp == 0
    per_w = S // n_sgrp
    assert per_w >= 4 and per_w % 3 == 2

    xi = x.astype(jnp.int32).reshape(B * S)

    mesh = plsc.VectorSubcoreMesh(core_axis_name="c", subcore_axis_name="s")

    cp = pltpu.CompilerParams()
    if "needs_layout_passes" in pltpu.CompilerParams.__dataclass_fields__:
        cp = dataclasses.replace(cp, needs_layout_passes=False)

    @functools.partial(
        pl.kernel,
        mesh=mesh,
        compiler_params=cp,
        out_type=jax.ShapeDtypeStruct((B, S * D), jnp.float32),
        scratch_types=[
            pltpu.VMEM((S, D), jnp.float32),
            pltpu.VMEM((_W * S,), jnp.int32),
            pltpu.VMEM((per_w, _W), jnp.int32),
            pltpu.VMEM((_W, D), jnp.float32),
            pltpu.VMEM((_W, D), jnp.float32),
            pltpu.VMEM((_W, D), jnp.float32),
            pltpu.SemaphoreType.DMA,
            pltpu.SemaphoreType.DMA,
            pltpu.SemaphoreType.DMA,
            pltpu.SemaphoreType.DMA,
            pltpu.SemaphoreType.DMA,
            pltpu.SemaphoreType.DMA,
        ],
    )
    def k(x_hbm, table_hbm, pe_hbm, out_hbm,
          pe_v, xb_v, idx_v, r0, r1, r2, g0, g1, g2, o0, o1, o2):
        rows = (r0, r1, r2)
        gsem = (g0, g1, g2)
        osem = (o0, o1, o2)
        wid = lax.axis_index("s") * _NC + lax.axis_index("c")
        bblk = wid % bblk_per_s
        s0 = (wid // bblk_per_s) * per_w
        b_lo = pl.multiple_of(bblk * _W, _W)

        pltpu.sync_copy(pe_hbm, pe_v)
        pltpu.sync_copy(x_hbm.at[pl.ds(b_lo * S, _W * S)], xb_v)

        iota_s = lax.iota(jnp.int32, _L) * S

        @pl.loop(0, per_w)
        def _(t):
            for j in range(_W // _L):
                vals = plsc.load_gather(xb_v, [iota_s + (_L * j * S + s0 + t)])
                idx_v[t, pl.ds(_L * j, _L)] = vals

        def gather(t, b):
            return pltpu.make_async_copy(
                table_hbm.at[idx_v.at[t]], rows[b], gsem[b])

        def out_slot(t):
            return out_hbm.at[
                pl.ds(b_lo, _W),
                pl.ds(pl.multiple_of((s0 + t) * D, D), D),
            ]

        def writeback(t, b):
            return pltpu.make_async_copy(rows[b], out_slot(t), osem[b])

        def compute(t, b):
            r = rows[b]
            pe_regs = [
                pe_v[s0 + t, pl.ds(cc * _L, _L)] for cc in range(D // _L)
            ]

            @pl.loop(0, _W)
            def _(i):
                for cc in range(D // _L):
                    sl = pl.ds(cc * _L, _L)
                    r[i, sl] = r[i, sl] * scale + pe_regs[cc]

        def body(t, b, issue_next, first=False):
            gather(t, b).wait()
            compute(t, b)
            pb = (b + 2) % 3

            def _wait_prev():
                writeback(t - 1, pb).wait()

            if first:
                pl.when(t >= 1)(_wait_prev)
            else:
                _wait_prev()

            if issue_next:
                gather(t + 2, pb).start()
            writeback(t, b).start()

        gather(0, 0).start()
        gather(1, 1).start()

        @pl.loop(0, per_w - 2, step=3)
        def _(t):
            body(t, 0, True, first=True)
            body(t + 1, 1, True)
            body(t + 2, 2, True)

        body(per_w - 2, (per_w - 2) % 3, False)
        body(per_w - 1, (per_w - 1) % 3, False)
        writeback(per_w - 1, (per_w - 1) % 3).wait()

    out = k(xi, table, pe)
    return out.reshape(B, S, D)


# speedup vs baseline: 1.7604x; 1.7371x over previous
import dataclasses
import functools

import jax
import jax.numpy as jnp
import numpy as np
from jax import lax
from jax.experimental import pallas as pl
from jax.experimental.pallas import tpu as pltpu
from jax.experimental.pallas import tpu_sc as plsc


def _positional_encoding(length: int, depth: int) -> np.ndarray:
    half = depth // 2
    positions = np.arange(length)[:, np.newaxis]
    depths = np.arange(half)[np.newaxis, :] / half
    angle_rates = 1.0 / (10000.0 ** depths)
    angle_rads = positions * angle_rates
    return np.concatenate(
        [np.sin(angle_rads), np.cos(angle_rads)], axis=-1
    ).astype(np.float32)


_NC, _NS, _L = 2, 16, 16
_NW = _NC * _NS
_W = 128
_CB, _CS = 16, 8


def kernel(x, table):
    B, S = x.shape
    V, D = table.shape
    scale = float(np.sqrt(float(D)))
    pe = jnp.asarray(_positional_encoding(S, D))

    assert B % _CB == 0 and S % _CS == 0 and D % _L == 0
    n_chunks = (B // _CB) * (S // _CS)
    assert n_chunks % _NW == 0
    per_w = n_chunks // _NW
    assert per_w >= 4 and per_w % 3 == 2
    n_sgrp = S // _CS
    bsub_per_w = per_w // n_sgrp
    rows_per_w = bsub_per_w * _CB

    xi = x.astype(jnp.int32)

    mesh = plsc.VectorSubcoreMesh(core_axis_name="c", subcore_axis_name="s")

    cp = pltpu.CompilerParams()
    if "needs_layout_passes" in pltpu.CompilerParams.__dataclass_fields__:
        cp = dataclasses.replace(cp, needs_layout_passes=False)

    @functools.partial(
        pl.kernel,
        mesh=mesh,
        compiler_params=cp,
        out_type=jax.ShapeDtypeStruct((B, S, D), jnp.float32),
        scratch_types=[
            pltpu.VMEM((S, D), jnp.float32),
            pltpu.VMEM((rows_per_w, S), jnp.int32),
            pltpu.VMEM((per_w, _W), jnp.int32),
            pltpu.VMEM((_W, D), jnp.float32),
            pltpu.VMEM((_W, D), jnp.float32),
            pltpu.VMEM((_W, D), jnp.float32),
            pltpu.SemaphoreType.DMA,
            pltpu.SemaphoreType.DMA,
            pltpu.SemaphoreType.DMA,
            pltpu.SemaphoreType.DMA,
            pltpu.SemaphoreType.DMA,
            pltpu.SemaphoreType.DMA,
        ],
    )
    def k(x_hbm, table_hbm, pe_hbm, out_hbm,
          pe_v, xb_v, idx_v, r0, r1, r2, g0, g1, g2, o0, o1, o2):
        rows = (r0, r1, r2)
        gsem = (g0, g1, g2)
        osem = (o0, o1, o2)
        wid = lax.axis_index("s") * _NC + lax.axis_index("c")
        b_lo = pl.multiple_of(wid * rows_per_w, rows_per_w)

        pltpu.sync_copy(pe_hbm, pe_v)
        pltpu.sync_copy(x_hbm.at[pl.ds(b_lo, rows_per_w)], xb_v)

        iota = lax.iota(jnp.int32, _L)
        bv = lax.shift_right_logical(iota, 3)
        sv = lax.bitwise_and(iota, 7)

        @pl.loop(0, per_w)
        def _(t):
            m = t // n_sgrp
            kk = t % n_sgrp
            for j in range(_W // _L):
                ridx = bv + (m * _CB + 2 * j)
                cidx = sv + kk * _CS
                vals = plsc.load_gather(xb_v, [ridx, cidx])
                idx_v[t, pl.ds(_L * j, _L)] = vals

        def gather(t, b):
            return pltpu.make_async_copy(
                table_hbm.at[idx_v.at[t]], rows[b], gsem[b])

        def out_slot(t):
            m = t // n_sgrp
            kk = t % n_sgrp
            return out_hbm.at[
                pl.ds(b_lo + m * _CB, _CB),
                pl.ds(pl.multiple_of(kk * _CS, _CS), _CS),
                slice(None),
            ]

        def writeback(t, b):
            return pltpu.make_async_copy(
                rows[b].reshape(_CB, _CS, D), out_slot(t), osem[b])

        def compute(t, b):
            kk = t % n_sgrp
            r = rows[b]
            for sp in range(_CS):
                pe_regs = [
                    pe_v[kk * _CS + sp, pl.ds(cc * _L, _L)]
                    for cc in range(D // _L)
                ]

                @pl.loop(0, _CB)
                def _(bp):
                    i = bp * _CS + sp
                    for cc in range(D // _L):
                        sl = pl.ds(cc * _L, _L)
                        r[i, sl] = r[i, sl] * scale + pe_regs[cc]

        def body(t, b, issue_next, first=False):
            gather(t, b).wait()
            compute(t, b)
            pb = (b + 2) % 3

            def _wait_prev():
                writeback(t - 1, pb).wait()

            if first:
                pl.when(t >= 1)(_wait_prev)
            else:
                _wait_prev()

            if issue_next:
                gather(t + 2, pb).start()
            writeback(t, b).start()

        gather(0, 0).start()
        gather(1, 1).start()

        @pl.loop(0, per_w - 2, step=3)
        def _(t):
            body(t, 0, True, first=True)
            body(t + 1, 1, True)
            body(t + 2, 2, True)

        body(per_w - 2, (per_w - 2) % 3, False)
        body(per_w - 1, (per_w - 1) % 3, False)
        writeback(per_w - 1, (per_w - 1) % 3).wait()

    return k(xi, table, pe)
